# baseline (speedup 1.0000x reference)
import functools

import jax
import jax.numpy as jnp
from jax import lax
from jax.experimental import pallas as pl
from jax.experimental.pallas import tpu as pltpu
from jax.experimental.pallas import tpu_sc as plsc

B, N, E = 8, 10000, 320000
NW = 32
QUARTERS = NW // B
EW = E // QUARTERS
C = 8000
NCH = EW // C
L = 16


def _prep_body(fp, fl, ft, t, v, ang, bce_o, cnt_o, sq_o, stab_o, ab_o):
    eps = 1e-7
    p = jnp.clip(fp[...], eps, 1.0 - eps)
    tl = fl[...]
    bce_o[...] = jnp.full(
        (1, 1), -jnp.sum(tl * jnp.log(p) + (1.0 - tl) * jnp.log(1.0 - p)))
    mask = tl > 0.5
    sq = jnp.where(mask, (ft[...] - t[...]) ** 2, 0.0)
    cnt_o[...] = jnp.full((1, 1), jnp.sum(mask.astype(jnp.float32)))
    sq_o[...] = jnp.full((1, 1), jnp.sum(sq))
    vv = v[...]
    low = jnp.maximum(0.95 - vv, 0.0)
    high = jnp.maximum(vv - 1.05, 0.0)
    stab_o[...] = jnp.full((1, 1), jnp.sum(low * low + high * high))
    th = ang[...]
    a = vv * jnp.cos(th)
    b = vv * jnp.sin(th)
    ua = ((jax.lax.bitcast_convert_type(a, jnp.uint32)
           + jnp.uint32(0x8000)) & jnp.uint32(0xFFFF0000))
    ub = (jax.lax.bitcast_convert_type(b, jnp.uint32)
          + jnp.uint32(0x8000)) >> jnp.uint32(16)
    ab_o[...] = jax.lax.bitcast_convert_type(ua | ub, jnp.int32)


_prep = pl.pallas_call(
    _prep_body,
    out_shape=(
        jax.ShapeDtypeStruct((1, 1), jnp.float32),
        jax.ShapeDtypeStruct((1, 1), jnp.float32),
        jax.ShapeDtypeStruct((1, 1), jnp.float32),
        jax.ShapeDtypeStruct((1, 1), jnp.float32),
        jax.ShapeDtypeStruct((B, N), jnp.int32),
    ),
)


def _unpack_hi(x):
    return plsc.bitcast(lax.bitwise_and(x, jnp.int32(-65536)), jnp.float32)


def _unpack_lo(x):
    return plsc.bitcast(lax.shift_left(x, jnp.int32(16)), jnp.float32)


def _pf_body(ab_hbm, sd_hbm, g_hbm, bs_hbm, lf_hbm, tl_hbm,
             pp_hbm, cap_hbm,
             ab_v, p_v, sd_v0, sd_v1, g_v0, g_v1,
             bs_v0, bs_v1, lf_v0, lf_v1, tl_v0, tl_v1, cap_v, sem0, sem1):
    wid = lax.axis_index("s") * 2 + lax.axis_index("c")
    batch = wid // QUARTERS
    quarter = wid % QUARTERS

    pltpu.sync_copy(ab_hbm.at[pl.ds(batch * N, N)], ab_v)

    @plsc.parallel_loop(0, N // L, unroll=8)
    def _(i):
        p_v[pl.ds(i * L, L)] = jnp.zeros((L,), jnp.float32)

    bufs = ((sd_v0, g_v0, bs_v0, lf_v0, tl_v0, sem0),
            (sd_v1, g_v1, bs_v1, lf_v1, tl_v1, sem1))

    def issue(ci, par):
        base = quarter * EW + ci * C
        ebase = batch * E + base
        sdv, gv, bv, lfv, tlv, sem = bufs[par]
        return [
            pltpu.async_copy(sd_hbm.at[pl.ds(base, C)], sdv, sem),
            pltpu.async_copy(g_hbm.at[pl.ds(ebase, C)], gv, sem),
            pltpu.async_copy(bs_hbm.at[pl.ds(ebase, C)], bv, sem),
            pltpu.async_copy(lf_hbm.at[pl.ds(ebase, C)], lfv, sem),
            pltpu.async_copy(tl_hbm.at[pl.ds(ebase, C)], tlv, sem),
        ]

    cap0 = jnp.zeros((L,), jnp.float32)
    pending = issue(0, 0)
    for ci in range(NCH):
        par = ci & 1
        sdv, gv, bv, lfv, tlv, _ = bufs[par]
        for h in pending:
            h.wait()
        if ci + 1 < NCH:
            pending = issue(ci + 1, 1 - par)

        @plsc.parallel_loop(0, C // L, unroll=5, carry=cap0)
        def cap0(i, acc):
            off = i * L
            sd = sdv[pl.ds(off, L)]
            s = lax.bitwise_and(sd, jnp.int32(0x3FFF))
            d = lax.shift_right_logical(sd, jnp.int32(14))
            g = gv[pl.ds(off, L)]
            bb = bv[pl.ds(off, L)]
            ab_s = plsc.load_gather(ab_v, [s])
            ab_d = plsc.load_gather(ab_v, [d])
            a_s = _unpack_hi(ab_s)
            b_s = _unpack_lo(ab_s)
            a_d = _unpack_hi(ab_d)
            b_d = _unpack_lo(ab_d)
            p = a_s * (g * a_d - bb * b_d) + b_s * (g * b_d + bb * a_d)
            plsc.addupdate_scatter(p_v, [s], p)
            plsc.addupdate_scatter(p_v, [d], -p)
            viol = jnp.maximum(lfv[pl.ds(off, L)] - tlv[pl.ds(off, L)], 0.0)
            return acc + viol * viol

    cap_v[...] = cap0
    pltpu.sync_copy(p_v, pp_hbm.at[pl.ds((quarter * B + batch) * N, N)])
    pltpu.sync_copy(cap_v, cap_hbm.at[pl.ds(wid * L, L)])


_pf = functools.partial(
    pl.kernel,
    mesh=plsc.VectorSubcoreMesh(core_axis_name="c", subcore_axis_name="s"),
    compiler_params=pltpu.CompilerParams(needs_layout_passes=False),
    out_type=(
        jax.ShapeDtypeStruct((QUARTERS * B * N,), jnp.float32),
        jax.ShapeDtypeStruct((NW * L,), jnp.float32),
    ),
    scratch_types=[
        pltpu.VMEM((N,), jnp.int32),
        pltpu.VMEM((N,), jnp.float32),
        pltpu.VMEM((C,), jnp.int32),
        pltpu.VMEM((C,), jnp.int32),
        pltpu.VMEM((C,), jnp.float32),
        pltpu.VMEM((C,), jnp.float32),
        pltpu.VMEM((C,), jnp.float32),
        pltpu.VMEM((C,), jnp.float32),
        pltpu.VMEM((C,), jnp.float32),
        pltpu.VMEM((C,), jnp.float32),
        pltpu.VMEM((C,), jnp.float32),
        pltpu.VMEM((C,), jnp.float32),
        pltpu.VMEM((L,), jnp.float32),
        pltpu.SemaphoreType.DMA,
        pltpu.SemaphoreType.DMA,
    ],
)(_pf_body)


def _fin_body(pp, pinj, capp, o, oc):
    p = pp[0] + pp[1] + pp[2] + pp[3]
    d = p - pinj[...]
    o[...] = jnp.full((1, 1), jnp.sum(d * d))
    oc[...] = jnp.full((1, 1), jnp.sum(capp[...]))


_fin = pl.pallas_call(
    _fin_body,
    out_shape=(
        jax.ShapeDtypeStruct((1, 1), jnp.float32),
        jax.ShapeDtypeStruct((1, 1), jnp.float32),
    ),
)


def kernel(failure_probability, failure_label, failure_timing, failure_time,
           voltages, angles, edge_index, conductance, susceptance,
           power_injection, line_flows, thermal_limits):
    v2 = voltages[..., 0]
    ang2 = angles[..., 0]
    t2 = failure_time[:, None]
    bce_s, cnt, sq_s, stab_s, ab2 = _prep(
        failure_probability, failure_label, failure_timing, t2, v2, ang2)

    ei = edge_index.astype(jnp.int32)
    sd = ei[0] | (ei[1] << 14)
    g1 = conductance[..., 0].reshape(-1)
    bs1 = susceptance[..., 0].reshape(-1)
    lf1 = line_flows[..., 0].reshape(-1)
    tl1 = thermal_limits[..., 0].reshape(-1)
    pp, capp = _pf(ab2.reshape(-1), sd, g1, bs1, lf1, tl1)
    pf_s, cap_s = _fin(pp.reshape(QUARTERS, B, N), power_injection[..., 0],
                       capp.reshape(NW, L))

    bn = jnp.float32(B * N)
    bce = bce_s[0, 0] / bn
    cnt0 = cnt[0, 0]
    l_timing = sq_s[0, 0] / jnp.maximum(cnt0, 1.0)
    l_pred = bce + jnp.where(cnt0 > 0, 0.5 * l_timing, 0.0)
    l_pf = pf_s[0, 0] / bn
    l_cap = cap_s[0, 0] / jnp.float32(B * E)
    l_stab = stab_s[0, 0] / bn
    l_temporal = jnp.float32(0.0)
    l_total = (l_pred + 0.1 * l_pf + 0.05 * l_cap + 0.05 * l_stab
               + 0.02 * l_temporal)
    return (l_total, l_pred, l_pf, l_cap, l_stab, l_temporal)


# speedup vs baseline: 1.0425x; 1.0425x over previous
import functools

import jax
import jax.numpy as jnp
from jax import lax
from jax.experimental import pallas as pl
from jax.experimental.pallas import tpu as pltpu
from jax.experimental.pallas import tpu_sc as plsc

B, N, E = 8, 10000, 320000
NW = 32
QUARTERS = NW // B
EW = E // QUARTERS
C = 8000
NCH = EW // C
L = 16


def _prep_body(fp, fl, ft, t, v, ang, bce_o, cnt_o, sq_o, stab_o, ab_o):
    eps = 1e-7
    p = jnp.clip(fp[...], eps, 1.0 - eps)
    tl = fl[...]
    bce_o[...] = jnp.full(
        (1, 1), -jnp.sum(tl * jnp.log(p) + (1.0 - tl) * jnp.log(1.0 - p)))
    mask = tl > 0.5
    sq = jnp.where(mask, (ft[...] - t[...]) ** 2, 0.0)
    cnt_o[...] = jnp.full((1, 1), jnp.sum(mask.astype(jnp.float32)))
    sq_o[...] = jnp.full((1, 1), jnp.sum(sq))
    vv = v[...]
    low = jnp.maximum(0.95 - vv, 0.0)
    high = jnp.maximum(vv - 1.05, 0.0)
    stab_o[...] = jnp.full((1, 1), jnp.sum(low * low + high * high))
    th = ang[...]
    a = vv * jnp.cos(th)
    b = vv * jnp.sin(th)
    ua = ((jax.lax.bitcast_convert_type(a, jnp.uint32)
           + jnp.uint32(0x8000)) & jnp.uint32(0xFFFF0000))
    ub = (jax.lax.bitcast_convert_type(b, jnp.uint32)
          + jnp.uint32(0x8000)) >> jnp.uint32(16)
    ab_o[...] = jax.lax.bitcast_convert_type(ua | ub, jnp.int32)


_prep = pl.pallas_call(
    _prep_body,
    out_shape=(
        jax.ShapeDtypeStruct((1, 1), jnp.float32),
        jax.ShapeDtypeStruct((1, 1), jnp.float32),
        jax.ShapeDtypeStruct((1, 1), jnp.float32),
        jax.ShapeDtypeStruct((1, 1), jnp.float32),
        jax.ShapeDtypeStruct((B, N), jnp.int32),
    ),
)


def _unpack_hi(x):
    return plsc.bitcast(lax.bitwise_and(x, jnp.int32(-65536)), jnp.float32)


def _unpack_lo(x):
    return plsc.bitcast(lax.shift_left(x, jnp.int32(16)), jnp.float32)


def _pf_body(ab_hbm, ei_hbm, g_hbm, bs_hbm, lf_hbm, tl_hbm,
             pp_hbm, cap_hbm,
             ab_v, p_v, src_v0, src_v1, dst_v0, dst_v1, g_v0, g_v1,
             bs_v0, bs_v1, lf_v0, lf_v1, tl_v0, tl_v1, cap_v, sem0, sem1):
    wid = lax.axis_index("s") * 2 + lax.axis_index("c")
    batch = wid // QUARTERS
    quarter = wid % QUARTERS

    pltpu.sync_copy(ab_hbm.at[pl.ds(batch * N, N)], ab_v)

    @plsc.parallel_loop(0, N // L, unroll=8)
    def _(i):
        p_v[pl.ds(i * L, L)] = jnp.zeros((L,), jnp.float32)

    bufs = ((src_v0, dst_v0, g_v0, bs_v0, lf_v0, tl_v0, sem0),
            (src_v1, dst_v1, g_v1, bs_v1, lf_v1, tl_v1, sem1))

    def issue(ci, par):
        base = quarter * EW + ci * C
        ebase = batch * E + base
        sv, dv, gv, bv, lfv, tlv, sem = bufs[par]
        return [
            pltpu.async_copy(ei_hbm.at[pl.ds(base, C)], sv, sem),
            pltpu.async_copy(ei_hbm.at[pl.ds(E + base, C)], dv, sem),
            pltpu.async_copy(g_hbm.at[pl.ds(ebase, C)], gv, sem),
            pltpu.async_copy(bs_hbm.at[pl.ds(ebase, C)], bv, sem),
            pltpu.async_copy(lf_hbm.at[pl.ds(ebase, C)], lfv, sem),
            pltpu.async_copy(tl_hbm.at[pl.ds(ebase, C)], tlv, sem),
        ]

    cap0 = jnp.zeros((L,), jnp.float32)
    pending = issue(0, 0)
    for ci in range(NCH):
        par = ci & 1
        sv, dv, gv, bv, lfv, tlv, _ = bufs[par]
        for h in pending:
            h.wait()
        if ci + 1 < NCH:
            pending = issue(ci + 1, 1 - par)

        @plsc.parallel_loop(0, C // L, unroll=5, carry=cap0)
        def cap0(i, acc):
            off = i * L
            s = sv[pl.ds(off, L)]
            d = dv[pl.ds(off, L)]
            g = gv[pl.ds(off, L)]
            bb = bv[pl.ds(off, L)]
            ab_s = plsc.load_gather(ab_v, [s])
            ab_d = plsc.load_gather(ab_v, [d])
            a_s = _unpack_hi(ab_s)
            b_s = _unpack_lo(ab_s)
            a_d = _unpack_hi(ab_d)
            b_d = _unpack_lo(ab_d)
            p = a_s * (g * a_d - bb * b_d) + b_s * (g * b_d + bb * a_d)
            plsc.addupdate_scatter(p_v, [s], p)
            plsc.addupdate_scatter(p_v, [d], -p)
            viol = jnp.maximum(lfv[pl.ds(off, L)] - tlv[pl.ds(off, L)], 0.0)
            return acc + viol * viol

    cap_v[...] = cap0
    pltpu.sync_copy(p_v, pp_hbm.at[pl.ds((quarter * B + batch) * N, N)])
    pltpu.sync_copy(cap_v, cap_hbm.at[pl.ds(wid * L, L)])


_pf = functools.partial(
    pl.kernel,
    mesh=plsc.VectorSubcoreMesh(core_axis_name="c", subcore_axis_name="s"),
    compiler_params=pltpu.CompilerParams(needs_layout_passes=False),
    out_type=(
        jax.ShapeDtypeStruct((QUARTERS * B * N,), jnp.float32),
        jax.ShapeDtypeStruct((NW * L,), jnp.float32),
    ),
    scratch_types=[
        pltpu.VMEM((N,), jnp.int32),
        pltpu.VMEM((N,), jnp.float32),
        pltpu.VMEM((C,), jnp.int32),
        pltpu.VMEM((C,), jnp.int32),
        pltpu.VMEM((C,), jnp.int32),
        pltpu.VMEM((C,), jnp.int32),
        pltpu.VMEM((C,), jnp.float32),
        pltpu.VMEM((C,), jnp.float32),
        pltpu.VMEM((C,), jnp.float32),
        pltpu.VMEM((C,), jnp.float32),
        pltpu.VMEM((C,), jnp.float32),
        pltpu.VMEM((C,), jnp.float32),
        pltpu.VMEM((C,), jnp.float32),
        pltpu.VMEM((C,), jnp.float32),
        pltpu.VMEM((L,), jnp.float32),
        pltpu.SemaphoreType.DMA,
        pltpu.SemaphoreType.DMA,
    ],
)(_pf_body)


def _fin_body(pp, pinj, capp, o, oc):
    p = pp[0] + pp[1] + pp[2] + pp[3]
    d = p - pinj[...]
    o[...] = jnp.full((1, 1), jnp.sum(d * d))
    oc[...] = jnp.full((1, 1), jnp.sum(capp[...]))


_fin = pl.pallas_call(
    _fin_body,
    out_shape=(
        jax.ShapeDtypeStruct((1, 1), jnp.float32),
        jax.ShapeDtypeStruct((1, 1), jnp.float32),
    ),
)


def kernel(failure_probability, failure_label, failure_timing, failure_time,
           voltages, angles, edge_index, conductance, susceptance,
           power_injection, line_flows, thermal_limits):
    v2 = voltages[..., 0]
    ang2 = angles[..., 0]
    t2 = failure_time[:, None]
    bce_s, cnt, sq_s, stab_s, ab2 = _prep(
        failure_probability, failure_label, failure_timing, t2, v2, ang2)

    ei = edge_index.astype(jnp.int32).reshape(-1)
    g1 = conductance[..., 0].reshape(-1)
    bs1 = susceptance[..., 0].reshape(-1)
    lf1 = line_flows[..., 0].reshape(-1)
    tl1 = thermal_limits[..., 0].reshape(-1)
    pp, capp = _pf(ab2.reshape(-1), ei, g1, bs1, lf1, tl1)
    pf_s, cap_s = _fin(pp.reshape(QUARTERS, B, N), power_injection[..., 0],
                       capp.reshape(NW, L))

    bn = jnp.float32(B * N)
    bce = bce_s[0, 0] / bn
    cnt0 = cnt[0, 0]
    l_timing = sq_s[0, 0] / jnp.maximum(cnt0, 1.0)
    l_pred = bce + jnp.where(cnt0 > 0, 0.5 * l_timing, 0.0)
    l_pf = pf_s[0, 0] / bn
    l_cap = cap_s[0, 0] / jnp.float32(B * E)
    l_stab = stab_s[0, 0] / bn
    l_temporal = jnp.float32(0.0)
    l_total = (l_pred + 0.1 * l_pf + 0.05 * l_cap + 0.05 * l_stab
               + 0.02 * l_temporal)
    return (l_total, l_pred, l_pf, l_cap, l_stab, l_temporal)
